# baseline (speedup 1.0000x reference)
import functools

import jax
import jax.numpy as jnp
import numpy as np
from jax.experimental import pallas as pl
from jax.experimental.pallas import tpu as pltpu

N_FFT = 512
HOP = 256
B = 64
L = 16384
E = 8
K = 2
FREQ = N_FFT // 2 + 1
FPAD = 384
NF = 1 + L // HOP
HP = jax.lax.Precision.HIGHEST
GP = jax.lax.Precision.HIGHEST
EP = jax.lax.Precision.DEFAULT


def _gate_body(F_ref, C_ref, S_ref, P_ref, gW1_ref, gb1_ref, gW2_ref, gb2_ref,
               gW3_ref, gb3_ref, idx_ref, w_ref):
    F = F_ref[:, :]
    Re = jnp.dot(F, C_ref[:, :], precision=GP)
    Im = jnp.dot(F, S_ref[:, :], precision=GP)
    mag = jnp.sqrt(Re * Re + Im * Im)
    pooled = jnp.dot(P_ref[:, :], mag, precision=GP)
    h = jax.nn.relu(jnp.dot(pooled, gW1_ref[:, :], precision=HP) + gb1_ref[0, :])
    h = jax.nn.relu(jnp.dot(h, gW2_ref[:, :], precision=HP) + gb2_ref[0, :])
    logits = jnp.dot(h, gW3_ref[:, :], precision=HP) + gb3_ref[0, :]
    iota = jax.lax.broadcasted_iota(jnp.int32, (B, 128), 1)
    m1 = jnp.max(logits, axis=1, keepdims=True)
    i1 = jnp.min(jnp.where(logits == m1, iota, 10000), axis=1, keepdims=True)
    l2 = jnp.where(iota == i1, -1e30, logits)
    m2 = jnp.max(l2, axis=1, keepdims=True)
    i2 = jnp.min(jnp.where(l2 == m2, iota, 10000), axis=1, keepdims=True)
    d = jnp.exp(m2 - m1)
    w1 = 1.0 / (1.0 + d)
    w2 = d / (1.0 + d)
    idx_ref[:, :] = jnp.concatenate([i1, i2], axis=1).astype(jnp.int32)
    w_ref[:, :] = jnp.concatenate([w1, w2], axis=1)


def _eval_expert(xc, W1_ref, b1_ref, W2A_ref, W2B_ref, b2_ref, W3A_ref,
                 W3B_ref, b3_ref):
    bf16 = jnp.bfloat16
    f32 = jnp.float32
    zcol = jnp.zeros((1, 128), bf16)
    h1g = jax.nn.relu(
        jnp.dot(xc, W1_ref[0], precision=EP, preferred_element_type=f32)
        + b1_ref[0]).astype(bf16)
    Ash = jnp.concatenate([h1g[1:, 0:128], zcol], axis=0)
    h2g = jax.nn.relu(
        jnp.dot(h1g, W2A_ref[0], precision=EP, preferred_element_type=f32)
        + jnp.dot(Ash, W2B_ref[0], precision=EP, preferred_element_type=f32)
        + b2_ref[0]).astype(bf16)
    A2 = jnp.concatenate([h2g[1:, 0:128], zcol], axis=0)
    return (jnp.dot(h2g, W3A_ref[0], precision=EP, preferred_element_type=f32)
            + jnp.dot(A2, W3B_ref[0], precision=EP, preferred_element_type=f32)
            + b3_ref[0])


def _expert_body(idx_ref, wv_ref, xc_ref,
                 W1_0, b1_0, W2A_0, W2B_0, b2_0, W3A_0, W3B_0, b3_0,
                 W1_1, b1_1, W2A_1, W2B_1, b2_1, W3A_1, W3B_1, b3_1,
                 out_ref):
    b = pl.program_id(0)
    xc = xc_ref[0]
    o0 = _eval_expert(xc, W1_0, b1_0, W2A_0, W2B_0, b2_0, W3A_0, W3B_0, b3_0)
    o1 = _eval_expert(xc, W1_1, b1_1, W2A_1, W2B_1, b2_1, W3A_1, W3B_1, b3_1)
    out_ref[0] = wv_ref[2 * b] * o0 + wv_ref[2 * b + 1] * o1


@functools.partial(jax.jit, static_argnames=())
def kernel(x, gW1, gb1, gW2, gb2, gW3, gb3, eW1, eb1, eW2, eb2, eW3, eb3):
    f32 = jnp.float32
    xp = jnp.pad(x, ((0, 0), (HOP, HOP)), mode='reflect')
    chunks = xp.reshape(B, NF + 1, HOP)
    frames = jnp.concatenate([chunks[:, :NF], chunks[:, 1:NF + 1]], axis=2)
    F = frames.reshape(B * NF, N_FFT)
    n = jnp.arange(N_FFT, dtype=f32)
    f = jnp.arange(FPAD, dtype=f32)
    hann = 0.5 * (1.0 - jnp.cos(2.0 * jnp.pi * n / N_FFT))
    ang = 2.0 * jnp.pi * n[:, None] * f[None, :] / N_FFT
    fmask = (f < FREQ).astype(f32)[None, :]
    Cm = (hann[:, None] * jnp.cos(ang)) * fmask
    Sm = (hann[:, None] * jnp.sin(ang)) * fmask
    P = (jnp.arange(B * NF) // NF == jnp.arange(B)[:, None]).astype(f32) / NF
    gW1p = jnp.pad(gW1, ((0, FPAD - FREQ), (0, 0)))
    gW3p = jnp.pad(gW3, ((0, 0), (0, 128 - E)))
    gb3p = jnp.pad(gb3, (0, 128 - E), constant_values=-1e30)

    vspec = lambda shp: pl.BlockSpec(shp, lambda: tuple(0 for _ in shp))
    idx, wgt = pl.pallas_call(
        _gate_body,
        out_shape=[jax.ShapeDtypeStruct((B, K), jnp.int32),
                   jax.ShapeDtypeStruct((B, K), f32)],
        in_specs=[vspec((B * NF, N_FFT)), vspec((N_FFT, FPAD)), vspec((N_FFT, FPAD)),
                  vspec((B, B * NF)), vspec((FPAD, 512)), vspec((1, 512)),
                  vspec((512, 256)), vspec((1, 256)), vspec((256, 128)),
                  vspec((1, 128))],
        out_specs=[vspec((B, K)), vspec((B, K))],
    )(F, Cm, Sm, P, gW1p, gb1[None, :], gW2, gb2[None, :], gW3p, gb3p[None, :])

    xs = jnp.concatenate([jnp.zeros((B, 1), f32), x[:, :-1]], axis=1)
    lo = xs.reshape(B, 512, 32)
    hi = jnp.concatenate([x[:, 31:], jnp.zeros((B, 31), f32)], axis=1).reshape(B, 512, 32)
    xc8 = jnp.concatenate([lo, hi], axis=2)

    Msel = np.zeros((64, 8, 7), np.float32)
    for r in range(8):
        for j in range(7):
            Msel[4 * r + j, r, j] = 1.0
    W1bd = jnp.einsum('mrj,ecj->emrc', jnp.asarray(Msel),
                      eW1[:, :, 0, :]).reshape(E, 64, 512)
    b1g = jnp.tile(eb1, (1, 8))[:, None, :]
    W2t = jnp.transpose(eW2, (0, 3, 2, 1))
    T2 = np.zeros((8, 5, 2), np.float32)
    for j in range(2):
        for jj in range(5):
            if 4 * j + jj < 8:
                T2[4 * j + jj, jj, j] = 1.0
    W2A = jnp.einsum('rtj,etio->erijo', jnp.asarray(T2), W2t).reshape(E, 512, 256)
    W2B = jnp.concatenate(
        [jnp.concatenate([jnp.zeros((E, 64, 128), f32), W2t[:, 4]], axis=2),
         jnp.zeros((E, 64, 256), f32)], axis=1)
    b2g = jnp.tile(eb2, (1, 2))[:, None, :]
    W3t = jnp.transpose(eW3, (0, 3, 2, 1))
    W3A = W3t[:, 0:2].reshape(E, 256, 128)
    W3B = W3t[:, 2]
    b3g = eb3[:, None, :]

    idx_flat = idx.reshape(B * K)
    wgt_flat = wgt.reshape(B * K)

    bf16 = jnp.bfloat16
    xc8 = xc8.astype(bf16)
    W1bd = W1bd.astype(bf16)
    W2A = W2A.astype(bf16)
    W2B = W2B.astype(bf16)
    W3A = W3A.astype(bf16)
    W3B = W3B.astype(bf16)

    def emap0(b, idx_r, w_r):
        return (idx_r[2 * b], 0, 0)

    def emap1(b, idx_r, w_r):
        return (idx_r[2 * b + 1], 0, 0)

    wspecs = lambda emap: [
        pl.BlockSpec((1, 64, 512), emap),
        pl.BlockSpec((1, 1, 512), emap),
        pl.BlockSpec((1, 512, 256), emap),
        pl.BlockSpec((1, 128, 256), emap),
        pl.BlockSpec((1, 1, 256), emap),
        pl.BlockSpec((1, 256, 128), emap),
        pl.BlockSpec((1, 128, 128), emap),
        pl.BlockSpec((1, 1, 128), emap),
    ]
    wargs = (W1bd, b1g, W2A, W2B, b2g, W3A, W3B, b3g)
    out = pl.pallas_call(
        _expert_body,
        grid_spec=pltpu.PrefetchScalarGridSpec(
            num_scalar_prefetch=2,
            grid=(B,),
            in_specs=[pl.BlockSpec((1, 512, 64), lambda b, i, w: (b, 0, 0))]
                     + wspecs(emap0) + wspecs(emap1),
            out_specs=pl.BlockSpec((1, 512, 128), lambda b, i, w: (b, 0, 0)),
        ),
        out_shape=jax.ShapeDtypeStruct((B, 512, 128), f32),
    )(idx_flat, wgt_flat, xc8, *wargs, *wargs)

    return jnp.transpose(out, (0, 2, 1))


# speedup vs baseline: 1.1829x; 1.1829x over previous
import functools

import jax
import jax.numpy as jnp
import numpy as np
from jax.experimental import pallas as pl
from jax.experimental.pallas import tpu as pltpu

N_FFT = 512
HOP = 256
B = 64
L = 16384
E = 8
K = 2
FREQ = N_FFT // 2 + 1
FPAD = 384
NF = 1 + L // HOP
HP = jax.lax.Precision.HIGHEST
GP = jax.lax.Precision.HIGHEST
EP = jax.lax.Precision.DEFAULT


def _gate_body(F_ref, C_ref, S_ref, P_ref, gW1_ref, gb1_ref, gW2_ref, gb2_ref,
               gW3_ref, gb3_ref, idx_ref, w_ref):
    F = F_ref[:, :]
    Re = jnp.dot(F, C_ref[:, :], precision=GP)
    Im = jnp.dot(F, S_ref[:, :], precision=GP)
    mag = jnp.sqrt(Re * Re + Im * Im)
    pooled = jnp.dot(P_ref[:, :], mag, precision=GP)
    h = jax.nn.relu(jnp.dot(pooled, gW1_ref[:, :], precision=HP) + gb1_ref[0, :])
    h = jax.nn.relu(jnp.dot(h, gW2_ref[:, :], precision=HP) + gb2_ref[0, :])
    logits = jnp.dot(h, gW3_ref[:, :], precision=HP) + gb3_ref[0, :]
    iota = jax.lax.broadcasted_iota(jnp.int32, (B, 128), 1)
    m1 = jnp.max(logits, axis=1, keepdims=True)
    i1 = jnp.min(jnp.where(logits == m1, iota, 10000), axis=1, keepdims=True)
    l2 = jnp.where(iota == i1, -1e30, logits)
    m2 = jnp.max(l2, axis=1, keepdims=True)
    i2 = jnp.min(jnp.where(l2 == m2, iota, 10000), axis=1, keepdims=True)
    d = jnp.exp(m2 - m1)
    w1 = 1.0 / (1.0 + d)
    w2 = d / (1.0 + d)
    idx_ref[:, :] = jnp.concatenate([i1, i2], axis=1).astype(jnp.int32)
    w_ref[:, :] = jnp.concatenate([w1, w2], axis=1)


def _eval_expert(xc, W1_ref, b1_ref, W2A_ref, W2B_ref, b2_ref, W3A_ref,
                 W3B_ref, b3_ref):
    bf16 = jnp.bfloat16
    f32 = jnp.float32
    zcol = jnp.zeros((1, 128), bf16)
    h1g = jax.nn.relu(
        jnp.dot(xc, W1_ref[0], precision=EP, preferred_element_type=f32)
        + b1_ref[0]).astype(bf16)
    Ash = jnp.concatenate([h1g[1:, 0:128], zcol], axis=0)
    W2A = W2A_ref[0]
    W2B = W2B_ref[0]
    j0 = (jnp.dot(h1g[:, 0:256], W2A, precision=EP, preferred_element_type=f32)
          + jnp.dot(h1g[:, 256:384], W2B, precision=EP, preferred_element_type=f32))
    j1 = (jnp.dot(h1g[:, 256:512], W2A, precision=EP, preferred_element_type=f32)
          + jnp.dot(Ash, W2B, precision=EP, preferred_element_type=f32))
    h2g = jax.nn.relu(
        jnp.concatenate([j0, j1], axis=1) + b2_ref[0]).astype(bf16)
    A2 = jnp.concatenate([h2g[1:, 0:128], zcol], axis=0)
    dn = (((0,), (1,)), ((), ()))
    return (jax.lax.dot_general(W3A_ref[0], h2g, dn, precision=EP,
                                preferred_element_type=f32)
            + jax.lax.dot_general(W3B_ref[0], A2, dn, precision=EP,
                                  preferred_element_type=f32)
            + b3_ref[0])


def _expert_body(idx_ref, wv_ref, xc_ref,
                 W1_0, b1_0, W2A_0, W2B_0, b2_0, W3A_0, W3B_0, b3_0,
                 W1_1, b1_1, W2A_1, W2B_1, b2_1, W3A_1, W3B_1, b3_1,
                 out_ref):
    b = pl.program_id(0)
    xc = xc_ref[0]
    o0 = _eval_expert(xc, W1_0, b1_0, W2A_0, W2B_0, b2_0, W3A_0, W3B_0, b3_0)
    o1 = _eval_expert(xc, W1_1, b1_1, W2A_1, W2B_1, b2_1, W3A_1, W3B_1, b3_1)
    out_ref[0] = wv_ref[2 * b] * o0 + wv_ref[2 * b + 1] * o1


@functools.partial(jax.jit, static_argnames=())
def kernel(x, gW1, gb1, gW2, gb2, gW3, gb3, eW1, eb1, eW2, eb2, eW3, eb3):
    f32 = jnp.float32
    xp = jnp.pad(x, ((0, 0), (HOP, HOP)), mode='reflect')
    chunks = xp.reshape(B, NF + 1, HOP)
    frames = jnp.concatenate([chunks[:, :NF], chunks[:, 1:NF + 1]], axis=2)
    F = frames.reshape(B * NF, N_FFT)
    n = jnp.arange(N_FFT, dtype=f32)
    f = jnp.arange(FPAD, dtype=f32)
    hann = 0.5 * (1.0 - jnp.cos(2.0 * jnp.pi * n / N_FFT))
    ang = 2.0 * jnp.pi * n[:, None] * f[None, :] / N_FFT
    fmask = (f < FREQ).astype(f32)[None, :]
    Cm = (hann[:, None] * jnp.cos(ang)) * fmask
    Sm = (hann[:, None] * jnp.sin(ang)) * fmask
    P = (jnp.arange(B * NF) // NF == jnp.arange(B)[:, None]).astype(f32) / NF
    gW1p = jnp.pad(gW1, ((0, FPAD - FREQ), (0, 0)))
    gW3p = jnp.pad(gW3, ((0, 0), (0, 128 - E)))
    gb3p = jnp.pad(gb3, (0, 128 - E), constant_values=-1e30)

    vspec = lambda shp: pl.BlockSpec(shp, lambda: tuple(0 for _ in shp))
    idx, wgt = pl.pallas_call(
        _gate_body,
        out_shape=[jax.ShapeDtypeStruct((B, K), jnp.int32),
                   jax.ShapeDtypeStruct((B, K), f32)],
        in_specs=[vspec((B * NF, N_FFT)), vspec((N_FFT, FPAD)), vspec((N_FFT, FPAD)),
                  vspec((B, B * NF)), vspec((FPAD, 512)), vspec((1, 512)),
                  vspec((512, 256)), vspec((1, 256)), vspec((256, 128)),
                  vspec((1, 128))],
        out_specs=[vspec((B, K)), vspec((B, K))],
    )(F, Cm, Sm, P, gW1p, gb1[None, :], gW2, gb2[None, :], gW3p, gb3p[None, :])

    xs = jnp.concatenate([jnp.zeros((B, 1), f32), x[:, :-1]], axis=1)
    lo = xs.reshape(B, 512, 32)
    hi = jnp.concatenate([x[:, 31:], jnp.zeros((B, 31), f32)], axis=1).reshape(B, 512, 32)
    xc8 = jnp.concatenate([lo, hi], axis=2)

    Msel = np.zeros((64, 8, 7), np.float32)
    for r in range(8):
        for j in range(7):
            Msel[4 * r + j, r, j] = 1.0
    W1bd = jnp.einsum('mrj,ecj->emrc', jnp.asarray(Msel),
                      eW1[:, :, 0, :]).reshape(E, 64, 512)
    b1g = jnp.tile(eb1, (1, 8))[:, None, :]
    W2t = jnp.transpose(eW2, (0, 3, 2, 1))
    W2A = W2t[:, 0:4].reshape(E, 256, 128)
    W2B = jnp.concatenate([W2t[:, 4], jnp.zeros((E, 64, 128), f32)], axis=1)
    b2g = jnp.tile(eb2, (1, 2))[:, None, :]
    W3t = jnp.transpose(eW3, (0, 3, 2, 1))
    W3A = W3t[:, 0:2].reshape(E, 256, 128)
    W3B = W3t[:, 2]
    b3g = eb3[:, :, None]

    idx_flat = idx.reshape(B * K)
    wgt_flat = wgt.reshape(B * K)

    bf16 = jnp.bfloat16
    xc8 = xc8.astype(bf16)
    W1bd = W1bd.astype(bf16)
    W2A = W2A.astype(bf16)
    W2B = W2B.astype(bf16)
    W3A = W3A.astype(bf16)
    W3B = W3B.astype(bf16)

    def emap0(b, idx_r, w_r):
        return (idx_r[2 * b], 0, 0)

    def emap1(b, idx_r, w_r):
        return (idx_r[2 * b + 1], 0, 0)

    wspecs = lambda emap: [
        pl.BlockSpec((1, 64, 512), emap),
        pl.BlockSpec((1, 1, 512), emap),
        pl.BlockSpec((1, 256, 128), emap),
        pl.BlockSpec((1, 128, 128), emap),
        pl.BlockSpec((1, 1, 256), emap),
        pl.BlockSpec((1, 256, 128), emap),
        pl.BlockSpec((1, 128, 128), emap),
        pl.BlockSpec((1, 128, 1), emap),
    ]
    wargs = (W1bd, b1g, W2A, W2B, b2g, W3A, W3B, b3g)
    out = pl.pallas_call(
        _expert_body,
        grid_spec=pltpu.PrefetchScalarGridSpec(
            num_scalar_prefetch=2,
            grid=(B,),
            in_specs=[pl.BlockSpec((1, 512, 64), lambda b, i, w: (b, 0, 0))]
                     + wspecs(emap0) + wspecs(emap1),
            out_specs=pl.BlockSpec((1, 128, 512), lambda b, i, w: (b, 0, 0)),
        ),
        out_shape=jax.ShapeDtypeStruct((B, 128, 512), f32),
    )(idx_flat, wgt_flat, xc8, *wargs, *wargs)

    return out


# speedup vs baseline: 1.3800x; 1.1666x over previous
import functools

import jax
import jax.numpy as jnp
import numpy as np
from jax.experimental import pallas as pl
from jax.experimental.pallas import tpu as pltpu

N_FFT = 512
HOP = 256
B = 64
L = 16384
E = 8
K = 2
FREQ = N_FFT // 2 + 1
FPAD = 384
NF = 1 + L // HOP
HP = jax.lax.Precision.HIGHEST
GP = jax.lax.Precision.HIGHEST
EP = jax.lax.Precision.DEFAULT


def _gate_body(F_ref, C_ref, S_ref, P_ref, gW1_ref, gb1_ref, gW2_ref, gb2_ref,
               gW3_ref, gb3_ref, idx_ref, w_ref):
    NR = B * (NF + 1)
    CH = F_ref[:, :]
    RU = jnp.dot(CH, C_ref[:, :], precision=GP)
    IU = jnp.dot(CH, S_ref[:, :], precision=GP)
    zrow = jnp.zeros((1, FPAD), jnp.float32)
    sgn = 1.0 - 2.0 * (jax.lax.broadcasted_iota(jnp.int32, (1, FPAD), 1) % 2
                       ).astype(jnp.float32)
    Re = RU + sgn * jnp.concatenate([RU[1:], zrow], axis=0)
    Im = IU + sgn * jnp.concatenate([IU[1:], zrow], axis=0)
    fio = jax.lax.broadcasted_iota(jnp.int32, (1, FPAD), 1)

    def winc(X, neg_edge0):
        e0 = (-X[:, 1:2]) if neg_edge0 else X[:, 1:2]
        XL = jnp.concatenate([e0, X[:, 0:FPAD - 1]], axis=1)
        XR = jnp.concatenate([X[:, 1:FPAD], jnp.zeros_like(X[:, 0:1])], axis=1)
        e256 = (-X[:, 255:256]) if neg_edge0 else X[:, 255:256]
        XR = jnp.where(fio == 256, e256, XR)
        return 0.5 * X - 0.25 * (XL + XR)

    Rw = winc(Re, False)
    Iw = winc(Im, True)
    mag = jnp.sqrt(Rw * Rw + Iw * Iw)
    pooled = jnp.dot(P_ref[:, :], mag, precision=GP)
    h = jax.nn.relu(jnp.dot(pooled, gW1_ref[:, :], precision=HP) + gb1_ref[0, :])
    h = jax.nn.relu(jnp.dot(h, gW2_ref[:, :], precision=HP) + gb2_ref[0, :])
    logits = jnp.dot(h, gW3_ref[:, :], precision=HP) + gb3_ref[0, :]
    iota = jax.lax.broadcasted_iota(jnp.int32, (B, 128), 1)
    m1 = jnp.max(logits, axis=1, keepdims=True)
    i1 = jnp.min(jnp.where(logits == m1, iota, 10000), axis=1, keepdims=True)
    l2 = jnp.where(iota == i1, -1e30, logits)
    m2 = jnp.max(l2, axis=1, keepdims=True)
    i2 = jnp.min(jnp.where(l2 == m2, iota, 10000), axis=1, keepdims=True)
    d = jnp.exp(m2 - m1)
    w1 = 1.0 / (1.0 + d)
    w2 = d / (1.0 + d)
    idx_ref[:, :] = jnp.concatenate([i1, i2], axis=1).astype(jnp.int32)
    w_ref[:, :] = jnp.concatenate([w1, w2], axis=1)


def _eval_expert(xc, W1_ref, b1_ref, W2A_ref, W2B_ref, b2_ref, W3A_ref,
                 W3B_ref, b3_ref):
    bf16 = jnp.bfloat16
    f32 = jnp.float32
    zcol = jnp.zeros((1, 128), bf16)
    h1g = jax.nn.relu(
        jnp.dot(xc, W1_ref[0], precision=EP, preferred_element_type=f32)
        + b1_ref[0]).astype(bf16)
    Ash = jnp.concatenate([h1g[1:, 0:128], zcol], axis=0)
    W2A = W2A_ref[0]
    W2B = W2B_ref[0]
    j0 = (jnp.dot(h1g[:, 0:256], W2A, precision=EP, preferred_element_type=f32)
          + jnp.dot(h1g[:, 256:384], W2B, precision=EP, preferred_element_type=f32))
    j1 = (jnp.dot(h1g[:, 256:512], W2A, precision=EP, preferred_element_type=f32)
          + jnp.dot(Ash, W2B, precision=EP, preferred_element_type=f32))
    h2g = jax.nn.relu(
        jnp.concatenate([j0, j1], axis=1) + b2_ref[0]).astype(bf16)
    A2 = jnp.concatenate([h2g[1:, 0:128], zcol], axis=0)
    dn = (((0,), (1,)), ((), ()))
    return (jax.lax.dot_general(W3A_ref[0], h2g, dn, precision=EP,
                                preferred_element_type=f32)
            + jax.lax.dot_general(W3B_ref[0], A2, dn, precision=EP,
                                  preferred_element_type=f32)
            + b3_ref[0])


def _expert_body(idx_ref, wv_ref, xc_ref,
                 W1_0, b1_0, W2A_0, W2B_0, b2_0, W3A_0, W3B_0, b3_0,
                 W1_1, b1_1, W2A_1, W2B_1, b2_1, W3A_1, W3B_1, b3_1,
                 out_ref):
    b = pl.program_id(0)
    xc = xc_ref[0]
    o0 = _eval_expert(xc, W1_0, b1_0, W2A_0, W2B_0, b2_0, W3A_0, W3B_0, b3_0)
    o1 = _eval_expert(xc, W1_1, b1_1, W2A_1, W2B_1, b2_1, W3A_1, W3B_1, b3_1)
    out_ref[0] = wv_ref[2 * b] * o0 + wv_ref[2 * b + 1] * o1


@functools.partial(jax.jit, static_argnames=())
def kernel(x, gW1, gb1, gW2, gb2, gW3, gb3, eW1, eb1, eW2, eb2, eW3, eb3):
    f32 = jnp.float32
    xp = jnp.pad(x, ((0, 0), (HOP, HOP)), mode='reflect')
    CH = xp.reshape(B * (NF + 1), HOP)
    m = jnp.arange(HOP, dtype=f32)
    f = jnp.arange(FPAD, dtype=f32)
    ang = 2.0 * jnp.pi * m[:, None] * f[None, :] / N_FFT
    fmask = (f < FREQ).astype(f32)[None, :]
    Cm = jnp.cos(ang) * fmask
    Sm = jnp.sin(ang) * fmask
    rows = jnp.arange(B * (NF + 1))
    P = ((rows // (NF + 1) == jnp.arange(B)[:, None])
         & (rows % (NF + 1) < NF)[None, :]).astype(f32) / NF
    gW1p = jnp.pad(gW1, ((0, FPAD - FREQ), (0, 0)))
    gW3p = jnp.pad(gW3, ((0, 0), (0, 128 - E)))
    gb3p = jnp.pad(gb3, (0, 128 - E), constant_values=-1e30)

    vspec = lambda shp: pl.BlockSpec(shp, lambda: tuple(0 for _ in shp))
    idx, wgt = pl.pallas_call(
        _gate_body,
        out_shape=[jax.ShapeDtypeStruct((B, K), jnp.int32),
                   jax.ShapeDtypeStruct((B, K), f32)],
        in_specs=[vspec((B * (NF + 1), HOP)), vspec((HOP, FPAD)), vspec((HOP, FPAD)),
                  vspec((B, B * (NF + 1))), vspec((FPAD, 512)), vspec((1, 512)),
                  vspec((512, 256)), vspec((1, 256)), vspec((256, 128)),
                  vspec((1, 128))],
        out_specs=[vspec((B, K)), vspec((B, K))],
    )(CH, Cm, Sm, P, gW1p, gb1[None, :], gW2, gb2[None, :], gW3p, gb3p[None, :])

    xs = jnp.concatenate([jnp.zeros((B, 1), f32), x[:, :-1]], axis=1)
    lo = xs.reshape(B, 512, 32)
    hi = jnp.concatenate([x[:, 31:], jnp.zeros((B, 31), f32)], axis=1).reshape(B, 512, 32)
    xc8 = jnp.concatenate([lo, hi], axis=2)

    Msel = np.zeros((64, 8, 7), np.float32)
    for r in range(8):
        for j in range(7):
            Msel[4 * r + j, r, j] = 1.0
    W1bd = jnp.einsum('mrj,ecj->emrc', jnp.asarray(Msel),
                      eW1[:, :, 0, :]).reshape(E, 64, 512)
    b1g = jnp.tile(eb1, (1, 8))[:, None, :]
    W2t = jnp.transpose(eW2, (0, 3, 2, 1))
    W2A = W2t[:, 0:4].reshape(E, 256, 128)
    W2B = jnp.concatenate([W2t[:, 4], jnp.zeros((E, 64, 128), f32)], axis=1)
    b2g = jnp.tile(eb2, (1, 2))[:, None, :]
    W3t = jnp.transpose(eW3, (0, 3, 2, 1))
    W3A = W3t[:, 0:2].reshape(E, 256, 128)
    W3B = W3t[:, 2]
    b3g = eb3[:, :, None]

    idx_flat = idx.reshape(B * K)
    wgt_flat = wgt.reshape(B * K)

    bf16 = jnp.bfloat16
    xc8 = xc8.astype(bf16)
    W1bd = W1bd.astype(bf16)
    W2A = W2A.astype(bf16)
    W2B = W2B.astype(bf16)
    W3A = W3A.astype(bf16)
    W3B = W3B.astype(bf16)

    def emap0(b, idx_r, w_r):
        return (idx_r[2 * b], 0, 0)

    def emap1(b, idx_r, w_r):
        return (idx_r[2 * b + 1], 0, 0)

    wspecs = lambda emap: [
        pl.BlockSpec((1, 64, 512), emap),
        pl.BlockSpec((1, 1, 512), emap),
        pl.BlockSpec((1, 256, 128), emap),
        pl.BlockSpec((1, 128, 128), emap),
        pl.BlockSpec((1, 1, 256), emap),
        pl.BlockSpec((1, 256, 128), emap),
        pl.BlockSpec((1, 128, 128), emap),
        pl.BlockSpec((1, 128, 1), emap),
    ]
    wargs = (W1bd, b1g, W2A, W2B, b2g, W3A, W3B, b3g)
    out = pl.pallas_call(
        _expert_body,
        grid_spec=pltpu.PrefetchScalarGridSpec(
            num_scalar_prefetch=2,
            grid=(B,),
            in_specs=[pl.BlockSpec((1, 512, 64), lambda b, i, w: (b, 0, 0))]
                     + wspecs(emap0) + wspecs(emap1),
            out_specs=pl.BlockSpec((1, 128, 512), lambda b, i, w: (b, 0, 0)),
        ),
        out_shape=jax.ShapeDtypeStruct((B, 128, 512), f32),
    )(idx_flat, wgt_flat, xc8, *wargs, *wargs)

    return out
